# baseline (speedup 1.0000x reference)
import functools

import jax
import jax.numpy as jnp
from jax.experimental import pallas as pl
from jax.experimental.pallas import tpu as pltpu

_S = 56
_BATCH = 256
_D = 30
_CELLS = _BATCH * _S * _S
_GROUPS = 64
_LANES = _GROUPS * _D
_ROWS = _CELLS // _GROUPS
_CORES = 2
_BLOCK_ROWS = 128
_STEPS = _ROWS // (_CORES * _BLOCK_ROWS)
_CHUNK = 8
_NCHUNK = _BLOCK_ROWS // _CHUNK


def _weights():
    lane = jnp.arange(_LANES)
    ch = lane % _D
    grp = lane // _D
    col = jnp.arange(128)
    seg = (grp[:, None] == col[None, :]).astype(jnp.float32)
    extra_e = jnp.where(ch < 2, 5.0,
                        jnp.where(ch < 4, 0.0,
                                  jnp.where(ch == 4, 0.5, 1.0)))
    w_e = seg * extra_e[:, None]
    w_e = w_e + ((col[None, :] == _GROUPS) & (ch[:, None] == 4)) * 0.5
    extra_v = jnp.where((ch == 2) | (ch == 3), 5.0, 0.0)
    w_v = seg * extra_v[:, None]
    g = seg * (ch[:, None] == 4)
    return (w_e.astype(jnp.bfloat16), w_v.astype(jnp.bfloat16),
            g.astype(jnp.bfloat16))


def _loss_kernel(p_ref, t_ref, we_ref, wv_ref, g_ref, o_ref):
    j = pl.program_id(1)
    we = we_ref[...]
    wv = wv_ref[...]
    g = g_ref[...]
    onehot64 = (jax.lax.broadcasted_iota(jnp.int32, (1, 128), 1)
                == _GROUPS).astype(jnp.float32)

    dn = (((1,), (0,)), ((), ()))
    acc = jnp.zeros((_CHUNK, 128), jnp.float32)
    for k in range(_NCHUNK):
        rows = pl.ds(k * _CHUNK, _CHUNK)
        p = p_ref[rows, :]
        t = t_ref[rows, :]
        u = p + t
        pt = p * t
        e = p - t
        e2 = e * e
        s = pt * jax.lax.rsqrt(pt + 1e-20)
        v = u - (s + s)
        x = jax.lax.dot_general(e2.astype(jnp.bfloat16), we, dn,
                                preferred_element_type=jnp.float32)
        x = x + jax.lax.dot_general(v.astype(jnp.bfloat16), wv, dn,
                                    preferred_element_type=jnp.float32)
        obj = jax.lax.dot_general(t.astype(jnp.bfloat16), g, dn,
                                  preferred_element_type=jnp.float32)
        acc = acc + x * (obj + onehot64)

    partial = jnp.sum(acc, axis=0, keepdims=True) * (1.0 / _BATCH)

    @pl.when(j == 0)
    def _init():
        o_ref[...] = jnp.zeros_like(o_ref)

    o_ref[0] += partial


def kernel(predictions, target):
    p2 = predictions.reshape(_ROWS, _LANES)
    t2 = target.reshape(_ROWS, _LANES)
    w_e, w_v, g = _weights()

    in_spec = pl.BlockSpec(
        (_BLOCK_ROWS, _LANES), lambda i, j: (i * _STEPS + j, 0))
    w_spec = pl.BlockSpec((_LANES, 128), lambda i, j: (0, 0))
    out_spec = pl.BlockSpec((1, 1, 128), lambda i, j: (i, 0, 0))

    partials = pl.pallas_call(
        _loss_kernel,
        grid=(_CORES, _STEPS),
        in_specs=[in_spec, in_spec, w_spec, w_spec, w_spec],
        out_specs=out_spec,
        out_shape=jax.ShapeDtypeStruct((_CORES, 1, 128), jnp.float32),
        compiler_params=pltpu.CompilerParams(
            dimension_semantics=("parallel", "arbitrary")),
        name="yolo_loss",
    )(p2, t2, w_e, w_v, g)

    return jnp.sum(partials)


# speedup vs baseline: 2.0050x; 2.0050x over previous
import numpy as np

import jax
import jax.numpy as jnp
from jax.experimental import pallas as pl
from jax.experimental.pallas import tpu as pltpu

_S = 56
_BATCH = 256
_D = 30
_GROUPS = 64
_CHUNK_LANES = _GROUPS * _D
_ROW_LANES = _S * _S * _D
_NCHUNK = _ROW_LANES // _CHUNK_LANES
_CORES = 2
_BLOCK_ROWS = 8
_STEPS = _BATCH // (_CORES * _BLOCK_ROWS)
_MROWS = _BLOCK_ROWS * _NCHUNK


def _np_weights():
    lane = np.arange(_CHUNK_LANES)
    ch = lane % _D
    grp = lane // _D
    col = np.arange(128)
    seg = (grp[:, None] == col[None, :]).astype(np.float32)
    extra = np.where(ch < 4, 5.0, np.where(ch == 4, 0.5, 1.0))
    w = seg * extra[:, None].astype(np.float32)
    w = w + ((col[None, :] == _GROUPS) & (ch[:, None] == 4)) * 0.5
    g = seg * (ch[:, None] == 4)
    return w, g


_W_NP, _G_NP = (a.astype(np.float32) for a in _np_weights())


def _loss_kernel(p_ref, t_ref, w_ref, g_ref, o_ref, d2s_ref, ts_ref):
    j = pl.program_id(1)
    onehot64 = (jax.lax.broadcasted_iota(jnp.int32, (1, 128), 1)
                == _GROUPS).astype(jnp.float32)
    chlane = jax.lax.broadcasted_iota(jnp.int32, (1, _CHUNK_LANES), 1) % _D
    is_wh = (chlane == 2) | (chlane == 3)

    for k in range(_NCHUNK):
        lanes = pl.ds(k * _CHUNK_LANES, _CHUNK_LANES)
        p = p_ref[:, lanes]
        t = t_ref[:, lanes]
        u = p + t
        pt = p * t
        e = p - t
        e2 = e * e
        s = pt * jax.lax.rsqrt(pt + 1e-20)
        v = u - (s + s)
        d2 = jnp.where(is_wh, v, e2)
        rows = pl.ds(k * _BLOCK_ROWS, _BLOCK_ROWS)
        d2s_ref[rows, :] = d2.astype(jnp.bfloat16)
        ts_ref[rows, :] = t.astype(jnp.bfloat16)

    dn = (((1,), (0,)), ((), ()))
    x = jax.lax.dot_general(d2s_ref[...], w_ref[...].astype(jnp.bfloat16),
                            dn, preferred_element_type=jnp.float32)
    obj = jax.lax.dot_general(ts_ref[...], g_ref[...].astype(jnp.bfloat16),
                              dn, preferred_element_type=jnp.float32)
    contrib = x * (obj + onehot64)
    partial = jnp.sum(contrib, axis=0, keepdims=True) * (1.0 / _BATCH)

    @pl.when(j == 0)
    def _init():
        o_ref[...] = jnp.zeros_like(o_ref)

    o_ref[0] += partial


def kernel(predictions, target):
    p2 = predictions.reshape(_BATCH, _ROW_LANES)
    t2 = target.reshape(_BATCH, _ROW_LANES)

    in_spec = pl.BlockSpec(
        (_BLOCK_ROWS, _ROW_LANES), lambda i, j: (i * _STEPS + j, 0))
    w_spec = pl.BlockSpec((_CHUNK_LANES, 128), lambda i, j: (0, 0))
    out_spec = pl.BlockSpec((1, 1, 128), lambda i, j: (i, 0, 0))

    partials = pl.pallas_call(
        _loss_kernel,
        grid=(_CORES, _STEPS),
        in_specs=[in_spec, in_spec, w_spec, w_spec],
        out_specs=out_spec,
        out_shape=jax.ShapeDtypeStruct((_CORES, 1, 128), jnp.float32),
        scratch_shapes=[
            pltpu.VMEM((_MROWS, _CHUNK_LANES), jnp.bfloat16),
            pltpu.VMEM((_MROWS, _CHUNK_LANES), jnp.bfloat16),
        ],
        compiler_params=pltpu.CompilerParams(
            dimension_semantics=("parallel", "arbitrary")),
        name="yolo_loss",
    )(p2, t2, jnp.asarray(_W_NP), jnp.asarray(_G_NP))

    return jnp.sum(partials)


# speedup vs baseline: 2.8682x; 1.4305x over previous
import numpy as np

import jax
import jax.numpy as jnp
from jax.experimental import pallas as pl
from jax.experimental.pallas import tpu as pltpu

_S = 56
_BATCH = 256
_D = 30
_GROUPS = _S
_ROW_LANES = _S * _D
_CORES = 2
_BLOCK_B = 8
_STEPS = _BATCH // (_CORES * _BLOCK_B)
_SUB = 8
_NSUB = _S // _SUB
_MROWS = _BLOCK_B * _S


def _np_weights():
    lane = np.arange(_ROW_LANES)
    ch = lane % _D
    grp = lane // _D
    col = np.arange(128)
    seg = (grp[:, None] == col[None, :]).astype(np.float32)
    extra = np.where(ch < 4, 5.0, np.where(ch == 4, 0.5, 1.0))
    w = seg * extra[:, None].astype(np.float32)
    w = w + ((col[None, :] == _GROUPS) & (ch[:, None] == 4)) * 0.5
    g = seg * (ch[:, None] == 4)
    return w.astype(np.float32), g.astype(np.float32)


_W_NP, _G_NP = _np_weights()


def _loss_kernel(p_ref, t_ref, w_ref, g_ref, o_ref, d2s_ref, ts_ref):
    j = pl.program_id(1)
    onehot = (jax.lax.broadcasted_iota(jnp.int32, (1, 128), 1)
              == _GROUPS).astype(jnp.float32)
    chlane = jax.lax.broadcasted_iota(jnp.int32, (1, _ROW_LANES), 1) % _D
    is_wh = (chlane == 2) | (chlane == 3)

    for b in range(_BLOCK_B):
        for r in range(_NSUB):
            rows = pl.ds(r * _SUB, _SUB)
            p = p_ref[b, rows, :]
            t = t_ref[b, rows, :]
            u = p + t
            pt = p * t
            e = p - t
            e2 = e * e
            s = pt * jax.lax.rsqrt(pt + 1e-20)
            v = u - (s + s)
            d2 = jnp.where(is_wh, v, e2)
            orows = pl.ds(b * _S + r * _SUB, _SUB)
            d2s_ref[orows, :] = d2.astype(jnp.bfloat16)
            ts_ref[orows, :] = t.astype(jnp.bfloat16)

    dn = (((1,), (0,)), ((), ()))
    x = jax.lax.dot_general(d2s_ref[...], w_ref[...].astype(jnp.bfloat16),
                            dn, preferred_element_type=jnp.float32)
    obj = jax.lax.dot_general(ts_ref[...], g_ref[...].astype(jnp.bfloat16),
                              dn, preferred_element_type=jnp.float32)
    contrib = x * (obj + onehot)
    partial = jnp.sum(contrib, axis=0, keepdims=True) * (1.0 / _BATCH)

    @pl.when(j == 0)
    def _init():
        o_ref[...] = jnp.zeros_like(o_ref)

    o_ref[0] += partial


def kernel(predictions, target):
    p2 = predictions.reshape(_BATCH, _S, _ROW_LANES)
    t2 = target.reshape(_BATCH, _S, _ROW_LANES)

    in_spec = pl.BlockSpec(
        (_BLOCK_B, _S, _ROW_LANES), lambda i, j: (i * _STEPS + j, 0, 0))
    w_spec = pl.BlockSpec((_ROW_LANES, 128), lambda i, j: (0, 0))
    out_spec = pl.BlockSpec((1, 1, 128), lambda i, j: (i, 0, 0))

    partials = pl.pallas_call(
        _loss_kernel,
        grid=(_CORES, _STEPS),
        in_specs=[in_spec, in_spec, w_spec, w_spec],
        out_specs=out_spec,
        out_shape=jax.ShapeDtypeStruct((_CORES, 1, 128), jnp.float32),
        scratch_shapes=[
            pltpu.VMEM((_MROWS, _ROW_LANES), jnp.bfloat16),
            pltpu.VMEM((_MROWS, _ROW_LANES), jnp.bfloat16),
        ],
        compiler_params=pltpu.CompilerParams(
            dimension_semantics=("parallel", "arbitrary")),
        name="yolo_loss",
    )(p2, t2, jnp.asarray(_W_NP), jnp.asarray(_G_NP))

    return jnp.sum(partials)


# speedup vs baseline: 19.4648x; 6.7865x over previous
import jax
import jax.numpy as jnp
from jax.experimental import pallas as pl
from jax.experimental.pallas import tpu as pltpu

_S = 56
_BATCH = 256
_D = 30
_CORES = 2
_BLOCK_S1 = 2
_STEPS = _S // (_CORES * _BLOCK_S1)
_LAMBDA_COORD = 5.0
_LAMBDA_NOOBJ = 0.5


def _plane_sq_diff(p_ref, t_ref, s1, ch):
    e = p_ref[s1, ch, :, :] - t_ref[s1, ch, :, :]
    return e * e


def _loss_kernel(p_ref, t_ref, o_ref):
    j = pl.program_id(1)

    acc = jnp.zeros((_S, _BATCH), jnp.float32)
    for s1 in range(_BLOCK_S1):
        x = jnp.zeros((_S, _BATCH), jnp.float32)
        for ch in (0, 1):
            x = x + _LAMBDA_COORD * _plane_sq_diff(p_ref, t_ref, s1, ch)
        for ch in (2, 3):
            p = p_ref[s1, ch, :, :]
            t = t_ref[s1, ch, :, :]
            pt = p * t
            s = pt * jax.lax.rsqrt(pt + 1e-20)
            x = x + _LAMBDA_COORD * (p + t - (s + s))
        for ch in range(5, _D):
            x = x + _plane_sq_diff(p_ref, t_ref, s1, ch)
        conf = _plane_sq_diff(p_ref, t_ref, s1, 4)
        x = x + _LAMBDA_NOOBJ * conf
        obj = (t_ref[s1, 4, :, :] == 1.0).astype(jnp.float32)
        acc = acc + obj * x + _LAMBDA_NOOBJ * conf

    partial = jnp.sum(acc, axis=0, keepdims=True) * (1.0 / _BATCH)

    @pl.when(j == 0)
    def _init():
        o_ref[...] = jnp.zeros_like(o_ref)

    o_ref[0] += partial


def kernel(predictions, target):
    pt_ = jnp.transpose(predictions, (1, 3, 2, 0))
    tt_ = jnp.transpose(target, (1, 3, 2, 0))

    in_spec = pl.BlockSpec(
        (_BLOCK_S1, _D, _S, _BATCH), lambda i, j: (i * _STEPS + j, 0, 0, 0))
    out_spec = pl.BlockSpec((1, 1, _BATCH), lambda i, j: (i, 0, 0))

    partials = pl.pallas_call(
        _loss_kernel,
        grid=(_CORES, _STEPS),
        in_specs=[in_spec, in_spec],
        out_specs=out_spec,
        out_shape=jax.ShapeDtypeStruct((_CORES, 1, _BATCH), jnp.float32),
        compiler_params=pltpu.CompilerParams(
            dimension_semantics=("parallel", "arbitrary")),
        name="yolo_loss",
    )(pt_, tt_)

    return jnp.sum(partials)
